# baseline (speedup 1.0000x reference)
import functools
import jax
import jax.numpy as jnp
from jax import lax
from jax.experimental import pallas as pl
from jax.experimental.pallas import tpu as pltpu
from jax.experimental.pallas import tpu_sc as plsc

_C = 256
_NG = _C // 16
_CH = 96
_HPC = _CH // 48


def _sc_body(x_hbm, t_hbm, h_hbm, w_hbm, out_hbm,
             hbuf, wbuf, tbuf, posb, buf0, buf1, sem0, sem1):
    cid = lax.axis_index("c")
    sid = lax.axis_index("s")
    wid = sid * 2 + cid
    t = wid % 16
    bbase = (wid // 16) * 4

    pltpu.sync_copy(h_hbm, hbuf)
    pltpu.sync_copy(w_hbm, wbuf)
    pltpu.sync_copy(t_hbm.at[pl.ds(t, 1)], tbuf)

    bufs = (buf0, buf1)
    sems = (sem0, sem1)

    def chunk_body(ci, carry):
        h0 = ci * _HPC
        row0 = ci * _CH

        def prow(r, c2):
            hr = h0 + r // 48
            wr = r % 48
            for g in range(_NG):
                sl = pl.ds(g * 16, 16)
                posb[r, sl] = tbuf[0, sl] + hbuf[hr, sl] + wbuf[wr, sl]
            return c2

        lax.fori_loop(0, _CH, prow, 0)

        def plane(b):
            return (bbase + b) * 16 + t

        handles = [None, None]
        handles[0] = pltpu.async_copy(
            x_hbm.at[plane(0), pl.ds(row0, _CH)], buf0, sem0)
        for b in range(4):
            cur = b % 2
            nxt = (b + 1) % 2
            if b < 3:
                handles[nxt] = pltpu.async_copy(
                    x_hbm.at[plane(b + 1), pl.ds(row0, _CH)],
                    bufs[nxt], sems[nxt])
            handles[cur].wait()

            def crow(r, c2, _buf=bufs[cur]):
                for g in range(_NG):
                    sl = pl.ds(g * 16, 16)
                    _buf[r, sl] = _buf[r, sl] + posb[r, sl]
                return c2

            lax.fori_loop(0, _CH, crow, 0)
            pltpu.sync_copy(bufs[cur], out_hbm.at[plane(b), pl.ds(row0, _CH)])
        return carry

    lax.fori_loop(0, 2304 // _CH, chunk_body, 0)


def kernel(x, T_embed, H_embed, W_embed):
    B, T, H, W, C = x.shape
    xf = x.reshape(B * T, H * W, C)
    mesh = plsc.VectorSubcoreMesh(core_axis_name="c", subcore_axis_name="s")
    run = functools.partial(
        pl.kernel,
        mesh=mesh,
        out_type=jax.ShapeDtypeStruct(xf.shape, xf.dtype),
        scratch_types=[
            pltpu.VMEM((H, C), jnp.float32),
            pltpu.VMEM((W, C), jnp.float32),
            pltpu.VMEM((1, C), jnp.float32),
            pltpu.VMEM((_CH, C), jnp.float32),
            pltpu.VMEM((_CH, C), jnp.float32),
            pltpu.VMEM((_CH, C), jnp.float32),
            pltpu.SemaphoreType.DMA,
            pltpu.SemaphoreType.DMA,
        ],
    )(_sc_body)
    out = run(xf, T_embed[:T], H_embed[:H], W_embed[:W])
    return out.reshape(B, T, H, W, C)


# speedup vs baseline: 3.0480x; 3.0480x over previous
import jax
import jax.numpy as jnp
from jax.experimental import pallas as pl
from jax.experimental.pallas import tpu as pltpu

_HB = 16


def _body(x_ref, t_ref, h_ref, w_ref, o_ref):
    t = t_ref[...]
    h = h_ref[...]
    w = w_ref[...]
    hw = h[:, None, :] + w[None, :, :]
    o_ref[0] = (x_ref[0] + t[:, None, None, :]) + hw[None, :, :, :]


def kernel(x, T_embed, H_embed, W_embed):
    B, T, H, W, C = x.shape
    return pl.pallas_call(
        _body,
        grid=(B, H // _HB),
        in_specs=[
            pl.BlockSpec((1, T, _HB, W, C), lambda b, hh: (b, 0, hh, 0, 0)),
            pl.BlockSpec((T, C), lambda b, hh: (0, 0)),
            pl.BlockSpec((_HB, C), lambda b, hh: (hh, 0)),
            pl.BlockSpec((W, C), lambda b, hh: (0, 0)),
        ],
        out_specs=pl.BlockSpec((1, T, _HB, W, C), lambda b, hh: (b, 0, hh, 0, 0)),
        out_shape=jax.ShapeDtypeStruct(x.shape, x.dtype),
        compiler_params=pltpu.CompilerParams(
            dimension_semantics=("arbitrary", "arbitrary"),
        ),
    )(x, T_embed[:T], H_embed[:H], W_embed[:W])


# speedup vs baseline: 3.0506x; 1.0009x over previous
import jax
import jax.numpy as jnp
from jax.experimental import pallas as pl
from jax.experimental.pallas import tpu as pltpu

_HB = 16


def _body(x_ref, t_ref, h_ref, w_ref, o_ref):
    hh = pl.program_id(1)
    t = t_ref[...]
    h = h_ref[pl.ds(hh * _HB, _HB), :]
    w = w_ref[...]
    hw = h[:, None, :] + w[None, :, :]
    o_ref[0] = (x_ref[0] + t[:, None, None, :]) + hw[None, :, :, :]


def kernel(x, T_embed, H_embed, W_embed):
    B, T, H, W, C = x.shape
    return pl.pallas_call(
        _body,
        grid=(B, H // _HB),
        in_specs=[
            pl.BlockSpec((1, T, _HB, W, C), lambda b, hh: (b, 0, hh, 0, 0)),
            pl.BlockSpec((T, C), lambda b, hh: (0, 0)),
            pl.BlockSpec((H, C), lambda b, hh: (0, 0)),
            pl.BlockSpec((W, C), lambda b, hh: (0, 0)),
        ],
        out_specs=pl.BlockSpec((1, T, _HB, W, C), lambda b, hh: (b, 0, hh, 0, 0)),
        out_shape=jax.ShapeDtypeStruct(x.shape, x.dtype),
        compiler_params=pltpu.CompilerParams(
            dimension_semantics=("parallel", "parallel"),
        ),
    )(x, T_embed[:T], H_embed[:H], W_embed[:W])
